# baseline (speedup 1.0000x reference)
import functools

import jax
import jax.numpy as jnp
from jax import lax
from jax.experimental import pallas as pl
from jax.experimental.pallas import tpu as pltpu
from jax.experimental.pallas import tpu_sc as plsc

_TN = 2048


def _sc_gather(table, idx):
    V, D = table.shape
    B = idx.shape[0]
    info = plsc.get_sparse_core_info()
    nw = info.num_cores * info.num_subcores
    b_per_w = B // nw
    mesh = plsc.VectorSubcoreMesh(core_axis_name="c", subcore_axis_name="s")

    @functools.partial(
        pl.kernel,
        mesh=mesh,
        compiler_params=pltpu.CompilerParams(use_tc_tiling_on_sc=False),
        out_type=jax.ShapeDtypeStruct((B, D), jnp.float32),
        scratch_types=[
            pltpu.VMEM((b_per_w,), jnp.int32),
            pltpu.VMEM((b_per_w, D), jnp.float32),
            pltpu.SemaphoreType.DMA,
        ],
    )
    def gather_kernel(table_hbm, idx_hbm, out_hbm, idx_v, rows_v, sem):
        wid = lax.axis_index("s") * info.num_cores + lax.axis_index("c")
        base = wid * b_per_w
        pltpu.sync_copy(idx_hbm.at[pl.ds(base, b_per_w)], idx_v)
        pltpu.async_copy(table_hbm.at[idx_v], rows_v, sem).wait()
        pltpu.sync_copy(rows_v, out_hbm.at[pl.ds(base, b_per_w)])

    return gather_kernel(table, idx)


def _gru_proj(e, carry, W_ir, b_ir, W_iz, b_iz, W_in, b_in,
              W_hr, W_hz, W_hn, b_hn, W_out, b_out):
    B, D = e.shape
    H = carry.shape[1]
    V = W_out.shape[1]
    grid = (pl.cdiv(V, _TN),)

    def body(e_ref, c_ref, wir, bir, wiz, biz, win, bin_, whr, whz, whn, bhn,
             wout, bout, logits_ref, newh_ref, h_s):
        @pl.when(pl.program_id(0) == 0)
        def _():
            e_ = e_ref[...]
            c = c_ref[...]
            hr = jnp.dot(c, whr[...], preferred_element_type=jnp.float32)
            hz = jnp.dot(c, whz[...], preferred_element_type=jnp.float32)
            hn = jnp.dot(c, whn[...], preferred_element_type=jnp.float32)
            r = jax.nn.sigmoid(
                jnp.dot(e_, wir[...], preferred_element_type=jnp.float32)
                + bir[...] + hr)
            z = jax.nn.sigmoid(
                jnp.dot(e_, wiz[...], preferred_element_type=jnp.float32)
                + biz[...] + hz)
            n = jnp.tanh(
                jnp.dot(e_, win[...], preferred_element_type=jnp.float32)
                + bin_[...] + r * (hn + bhn[...]))
            h = (1.0 - z) * n + z * c
            h_s[...] = h
            newh_ref[...] = h

        logits_ref[...] = (
            jnp.dot(h_s[...], wout[...], preferred_element_type=jnp.float32)
            + bout[...])

    const = lambda shape: pl.BlockSpec(shape, lambda i: (0,) * len(shape))
    out = pl.pallas_call(
        body,
        grid=grid,
        in_specs=[
            const((B, D)),
            const((B, H)),
            const((D, H)), const((1, H)),
            const((D, H)), const((1, H)),
            const((D, H)), const((1, H)),
            const((H, H)), const((H, H)), const((H, H)),
            const((1, H)),
            pl.BlockSpec((H, _TN), lambda i: (0, i)),
            pl.BlockSpec((1, _TN), lambda i: (0, i)),
        ],
        out_specs=[
            pl.BlockSpec((B, _TN), lambda i: (0, i)),
            const((B, H)),
        ],
        out_shape=[
            jax.ShapeDtypeStruct((B, V), jnp.float32),
            jax.ShapeDtypeStruct((B, H), jnp.float32),
        ],
        scratch_shapes=[pltpu.VMEM((B, H), jnp.float32)],
        compiler_params=pltpu.CompilerParams(
            dimension_semantics=("arbitrary",)),
    )(e, carry, W_ir, b_ir.reshape(1, H), W_iz, b_iz.reshape(1, H),
      W_in, b_in.reshape(1, H), W_hr, W_hz, W_hn, b_hn.reshape(1, H),
      W_out, b_out.reshape(1, V))
    return out[0], out[1]


def kernel(x, carry, embed_table, W_ir, b_ir, W_iz, b_iz, W_in, b_in,
           W_hr, W_hz, W_hn, b_hn, W_out, b_out):
    e = _sc_gather(embed_table, x)
    logits, new_h = _gru_proj(e, carry, W_ir, b_ir, W_iz, b_iz, W_in, b_in,
                              W_hr, W_hz, W_hn, b_hn, W_out, b_out)
    return (logits, new_h)


# speedup vs baseline: 1.0017x; 1.0017x over previous
import functools

import jax
import jax.numpy as jnp
from jax import lax
from jax.experimental import pallas as pl
from jax.experimental.pallas import tpu as pltpu
from jax.experimental.pallas import tpu_sc as plsc

_TN = 2048


def _sc_gather_pairs(table2, idx2):
    _, D2 = table2.shape
    B = idx2.shape[0]
    info = plsc.get_sparse_core_info()
    nw = info.num_cores * info.num_subcores
    b_per_w = B // nw
    mesh = plsc.VectorSubcoreMesh(core_axis_name="c", subcore_axis_name="s")

    @functools.partial(
        pl.kernel,
        mesh=mesh,
        out_type=jax.ShapeDtypeStruct((B, D2), jnp.float32),
        scratch_types=[
            pltpu.VMEM((b_per_w,), jnp.int32),
            pltpu.VMEM((b_per_w, D2), jnp.float32),
            pltpu.SemaphoreType.DMA,
        ],
    )
    def gather_kernel(table_hbm, idx_hbm, out_hbm, idx_v, rows_v, sem):
        wid = lax.axis_index("s") * info.num_cores + lax.axis_index("c")
        base = wid * b_per_w
        pltpu.sync_copy(idx_hbm.at[pl.ds(base, b_per_w)], idx_v)
        pltpu.async_copy(table_hbm.at[idx_v], rows_v, sem).wait()
        pltpu.sync_copy(rows_v, out_hbm.at[pl.ds(base, b_per_w)])

    return gather_kernel(table2, idx2)


def _gru_proj(e2, par, carry, W_ir, b_ir, W_iz, b_iz, W_in, b_in,
              W_hr, W_hz, W_hn, b_hn, W_out, b_out):
    B, D2 = e2.shape
    D = D2 // 2
    H = carry.shape[1]
    V = W_out.shape[1]
    grid = (pl.cdiv(V, _TN),)

    def body(e2_ref, par_ref, c_ref, wir, bir, wiz, biz, win, bin_, whr, whz,
             whn, bhn, wout, bout, logits_ref, newh_ref, h_s):
        @pl.when(pl.program_id(0) == 0)
        def _():
            p = par_ref[...]
            e_ = e2_ref[:, :D] * (1.0 - p) + e2_ref[:, D:] * p
            c = c_ref[...]
            hr = jnp.dot(c, whr[...], preferred_element_type=jnp.float32)
            hz = jnp.dot(c, whz[...], preferred_element_type=jnp.float32)
            hn = jnp.dot(c, whn[...], preferred_element_type=jnp.float32)
            r = jax.nn.sigmoid(
                jnp.dot(e_, wir[...], preferred_element_type=jnp.float32)
                + bir[...] + hr)
            z = jax.nn.sigmoid(
                jnp.dot(e_, wiz[...], preferred_element_type=jnp.float32)
                + biz[...] + hz)
            n = jnp.tanh(
                jnp.dot(e_, win[...], preferred_element_type=jnp.float32)
                + bin_[...] + r * (hn + bhn[...]))
            h = (1.0 - z) * n + z * c
            h_s[...] = h
            newh_ref[...] = h

        logits_ref[...] = (
            jnp.dot(h_s[...], wout[...], preferred_element_type=jnp.float32)
            + bout[...])

    const = lambda shape: pl.BlockSpec(shape, lambda i: (0,) * len(shape))
    out = pl.pallas_call(
        body,
        grid=grid,
        in_specs=[
            const((B, D2)),
            const((B, 1)),
            const((B, H)),
            const((D, H)), const((1, H)),
            const((D, H)), const((1, H)),
            const((D, H)), const((1, H)),
            const((H, H)), const((H, H)), const((H, H)),
            const((1, H)),
            pl.BlockSpec((H, _TN), lambda i: (0, i)),
            pl.BlockSpec((1, _TN), lambda i: (0, i)),
        ],
        out_specs=[
            pl.BlockSpec((B, _TN), lambda i: (0, i)),
            const((B, H)),
        ],
        out_shape=[
            jax.ShapeDtypeStruct((B, V), jnp.float32),
            jax.ShapeDtypeStruct((B, H), jnp.float32),
        ],
        scratch_shapes=[pltpu.VMEM((B, H), jnp.float32)],
        compiler_params=pltpu.CompilerParams(
            dimension_semantics=("arbitrary",)),
    )(e2, par, carry, W_ir, b_ir.reshape(1, H), W_iz, b_iz.reshape(1, H),
      W_in, b_in.reshape(1, H), W_hr, W_hz, W_hn, b_hn.reshape(1, H),
      W_out, b_out.reshape(1, V))
    return out[0], out[1]


def kernel(x, carry, embed_table, W_ir, b_ir, W_iz, b_iz, W_in, b_in,
           W_hr, W_hz, W_hn, b_hn, W_out, b_out):
    V, D = embed_table.shape
    table2 = embed_table.reshape(V // 2, 2 * D)
    e2 = _sc_gather_pairs(table2, x // 2)
    par = (x % 2).astype(jnp.float32).reshape(-1, 1)
    logits, new_h = _gru_proj(e2, par, carry, W_ir, b_ir, W_iz, b_iz,
                              W_in, b_in, W_hr, W_hz, W_hn, b_hn,
                              W_out, b_out)
    return (logits, new_h)


# speedup vs baseline: 2.7841x; 2.7794x over previous
import functools

import jax
import jax.numpy as jnp
from jax import lax
from jax.experimental import pallas as pl
from jax.experimental.pallas import tpu as pltpu
from jax.experimental.pallas import tpu_sc as plsc

_TM = 2048


def _sc_gather_pairs(table2, idx2):
    _, D2 = table2.shape
    B = idx2.shape[0]
    info = plsc.get_sparse_core_info()
    nw = info.num_cores * info.num_subcores
    b_per_w = B // nw
    mesh = plsc.VectorSubcoreMesh(core_axis_name="c", subcore_axis_name="s")

    @functools.partial(
        pl.kernel,
        mesh=mesh,
        out_type=jax.ShapeDtypeStruct((B, D2), jnp.float32),
        scratch_types=[
            pltpu.VMEM((b_per_w,), jnp.int32),
            pltpu.VMEM((b_per_w, D2), jnp.float32),
            pltpu.SemaphoreType.DMA,
        ],
    )
    def gather_kernel(table_hbm, idx_hbm, out_hbm, idx_v, rows_v, sem):
        wid = lax.axis_index("s") * info.num_cores + lax.axis_index("c")
        base = wid * b_per_w
        pltpu.sync_copy(idx_hbm.at[pl.ds(base, b_per_w)], idx_v)
        pltpu.async_copy(table_hbm.at[idx_v], rows_v, sem).wait()
        pltpu.sync_copy(rows_v, out_hbm.at[pl.ds(base, b_per_w)])

    return gather_kernel(table2, idx2)


def _gru_proj_t(e2, par, carry, W_ir, b_ir, W_iz, b_iz, W_in, b_in,
                W_hr, W_hz, W_hn, b_hn, WT, b_out):
    B, D2 = e2.shape
    D = D2 // 2
    H = carry.shape[1]
    V = WT.shape[0]
    grid = (pl.cdiv(V, _TM),)

    def body(e2_ref, par_ref, c_ref, wir, bir, wiz, biz, win, bin_, whr, whz,
             whn, bhn, wt, bout, logitsT_ref, newh_ref, h_s, hT_s):
        @pl.when(pl.program_id(0) == 0)
        def _():
            p = par_ref[...]
            e_ = e2_ref[:, :D] * (1.0 - p) + e2_ref[:, D:] * p
            c = c_ref[...]
            hr = jnp.dot(c, whr[...], preferred_element_type=jnp.float32)
            hz = jnp.dot(c, whz[...], preferred_element_type=jnp.float32)
            hn = jnp.dot(c, whn[...], preferred_element_type=jnp.float32)
            r = jax.nn.sigmoid(
                jnp.dot(e_, wir[...], preferred_element_type=jnp.float32)
                + bir[...] + hr)
            z = jax.nn.sigmoid(
                jnp.dot(e_, wiz[...], preferred_element_type=jnp.float32)
                + biz[...] + hz)
            n = jnp.tanh(
                jnp.dot(e_, win[...], preferred_element_type=jnp.float32)
                + bin_[...] + r * (hn + bhn[...]))
            h = (1.0 - z) * n + z * c
            h_s[...] = h
            hT_s[...] = h.T
            newh_ref[...] = h

        logitsT_ref[...] = (
            jnp.dot(wt[...], hT_s[...], preferred_element_type=jnp.float32)
            + bout[...].T)

    const = lambda shape: pl.BlockSpec(shape, lambda i: (0,) * len(shape))
    out = pl.pallas_call(
        body,
        grid=grid,
        in_specs=[
            const((B, D2)),
            const((B, 1)),
            const((B, H)),
            const((D, H)), const((1, H)),
            const((D, H)), const((1, H)),
            const((D, H)), const((1, H)),
            const((H, H)), const((H, H)), const((H, H)),
            const((1, H)),
            pl.BlockSpec((_TM, H), lambda i: (i, 0)),
            pl.BlockSpec((1, _TM), lambda i: (0, i)),
        ],
        out_specs=[
            pl.BlockSpec((_TM, B), lambda i: (i, 0)),
            const((B, H)),
        ],
        out_shape=[
            jax.ShapeDtypeStruct((V, B), jnp.float32),
            jax.ShapeDtypeStruct((B, H), jnp.float32),
        ],
        scratch_shapes=[pltpu.VMEM((B, H), jnp.float32),
                        pltpu.VMEM((H, B), jnp.float32)],
        compiler_params=pltpu.CompilerParams(
            dimension_semantics=("arbitrary",)),
    )(e2, par, carry, W_ir, b_ir.reshape(1, H), W_iz, b_iz.reshape(1, H),
      W_in, b_in.reshape(1, H), W_hr, W_hz, W_hn, b_hn.reshape(1, H),
      WT, b_out.reshape(1, V))
    return out[0], out[1]


def kernel(x, carry, embed_table, W_ir, b_ir, W_iz, b_iz, W_in, b_in,
           W_hr, W_hz, W_hn, b_hn, W_out, b_out):
    V, D = embed_table.shape
    table2 = embed_table.reshape(V // 2, 2 * D)
    e2 = _sc_gather_pairs(table2, x // 2)
    par = (x % 2).astype(jnp.float32).reshape(-1, 1)
    logitsT, new_h = _gru_proj_t(e2, par, carry, W_ir, b_ir, W_iz, b_iz,
                                 W_in, b_in, W_hr, W_hz, W_hn, b_hn,
                                 W_out.T, b_out)
    return (logitsT.T, new_h)


# speedup vs baseline: 3.1611x; 1.1354x over previous
import functools

import jax
import jax.numpy as jnp
from jax import lax
from jax.experimental import pallas as pl
from jax.experimental.pallas import tpu as pltpu
from jax.experimental.pallas import tpu_sc as plsc

_TM = 2048
_SPLIT = 50048
_TT = 2176


def _halves_table(aT):
    D, V = aT.shape
    nblk = _SPLIT // _TT

    def body(a1_ref, a2_ref, out_ref):
        out_ref[...] = jnp.concatenate(
            [a1_ref[...].T, a2_ref[...].T], axis=1)

    return pl.pallas_call(
        body,
        grid=(nblk,),
        in_specs=[
            pl.BlockSpec((D, _TT), lambda i: (0, i)),
            pl.BlockSpec((D, _TT), lambda i: (0, i + nblk)),
        ],
        out_specs=pl.BlockSpec((_TT, 2 * D), lambda i: (i, 0)),
        out_shape=jax.ShapeDtypeStruct((_SPLIT, 2 * D), jnp.float32),
        compiler_params=pltpu.CompilerParams(
            dimension_semantics=("arbitrary",)),
    )(aT, aT)


def _sc_gather(table2, idx):
    _, D2 = table2.shape
    B = idx.shape[0]
    info = plsc.get_sparse_core_info()
    nw = info.num_cores * info.num_subcores
    b_per_w = B // nw
    mesh = plsc.VectorSubcoreMesh(core_axis_name="c", subcore_axis_name="s")

    @functools.partial(
        pl.kernel,
        mesh=mesh,
        out_type=jax.ShapeDtypeStruct((B, D2), jnp.float32),
        scratch_types=[
            pltpu.VMEM((b_per_w,), jnp.int32),
            pltpu.VMEM((b_per_w, D2), jnp.float32),
            pltpu.SemaphoreType.DMA,
        ],
    )
    def gather_kernel(table_hbm, idx_hbm, out_hbm, idx_v, rows_v, sem):
        wid = lax.axis_index("s") * info.num_cores + lax.axis_index("c")
        base = wid * b_per_w
        pltpu.sync_copy(idx_hbm.at[pl.ds(base, b_per_w)], idx_v)
        pltpu.async_copy(table_hbm.at[idx_v], rows_v, sem).wait()
        pltpu.sync_copy(rows_v, out_hbm.at[pl.ds(base, b_per_w)])

    return gather_kernel(table2, idx)


def _gru_proj_t(e2, par, carry, W_ir, b_ir, W_iz, b_iz, W_in, b_in,
                W_hr, W_hz, W_hn, b_hn, WT, b_out):
    B, D2 = e2.shape
    D = D2 // 2
    H = carry.shape[1]
    V = WT.shape[0]
    grid = (pl.cdiv(V, _TM),)

    def body(e2_ref, par_ref, c_ref, wir, bir, wiz, biz, win, bin_, whr, whz,
             whn, bhn, wt, bout, logitsT_ref, newh_ref, h_s, hT_s):
        @pl.when(pl.program_id(0) == 0)
        def _():
            p = par_ref[...]
            e_ = jnp.where(p == 0.0, e2_ref[:, :D], e2_ref[:, D:])
            c = c_ref[...]
            hr = jnp.dot(c, whr[...], preferred_element_type=jnp.float32)
            hz = jnp.dot(c, whz[...], preferred_element_type=jnp.float32)
            hn = jnp.dot(c, whn[...], preferred_element_type=jnp.float32)
            r = jax.nn.sigmoid(
                jnp.dot(e_, wir[...], preferred_element_type=jnp.float32)
                + bir[...] + hr)
            z = jax.nn.sigmoid(
                jnp.dot(e_, wiz[...], preferred_element_type=jnp.float32)
                + biz[...] + hz)
            n = jnp.tanh(
                jnp.dot(e_, win[...], preferred_element_type=jnp.float32)
                + bin_[...] + r * (hn + bhn[...]))
            h = (1.0 - z) * n + z * c
            h_s[...] = h
            hT_s[...] = h.T
            newh_ref[...] = h

        logitsT_ref[...] = (
            jnp.dot(wt[...], hT_s[...], preferred_element_type=jnp.float32)
            + bout[...].T)

    const = lambda shape: pl.BlockSpec(shape, lambda i: (0,) * len(shape))
    out = pl.pallas_call(
        body,
        grid=grid,
        in_specs=[
            const((B, D2)),
            const((B, 1)),
            const((B, H)),
            const((D, H)), const((1, H)),
            const((D, H)), const((1, H)),
            const((D, H)), const((1, H)),
            const((H, H)), const((H, H)), const((H, H)),
            const((1, H)),
            pl.BlockSpec((_TM, H), lambda i: (i, 0)),
            pl.BlockSpec((1, _TM), lambda i: (0, i)),
        ],
        out_specs=[
            pl.BlockSpec((_TM, B), lambda i: (i, 0)),
            const((B, H)),
        ],
        out_shape=[
            jax.ShapeDtypeStruct((V, B), jnp.float32),
            jax.ShapeDtypeStruct((B, H), jnp.float32),
        ],
        scratch_shapes=[pltpu.VMEM((B, H), jnp.float32),
                        pltpu.VMEM((H, B), jnp.float32)],
        compiler_params=pltpu.CompilerParams(
            dimension_semantics=("arbitrary",)),
    )(e2, par, carry, W_ir, b_ir.reshape(1, H), W_iz, b_iz.reshape(1, H),
      W_in, b_in.reshape(1, H), W_hr, W_hz, W_hn, b_hn.reshape(1, H),
      WT, b_out.reshape(1, V))
    return out[0], out[1]


def kernel(x, carry, embed_table, W_ir, b_ir, W_iz, b_iz, W_in, b_in,
           W_hr, W_hz, W_hn, b_hn, W_out, b_out):
    table2 = _halves_table(embed_table.T)
    e2 = _sc_gather(table2, x % _SPLIT)
    par = (x >= _SPLIT).astype(jnp.float32).reshape(-1, 1)
    logitsT, new_h = _gru_proj_t(e2, par, carry, W_ir, b_ir, W_iz, b_iz,
                                 W_in, b_in, W_hr, W_hz, W_hn, b_hn,
                                 W_out.T, b_out)
    return (logitsT.T, new_h)


# speedup vs baseline: 3.2979x; 1.0433x over previous
import functools

import jax
import jax.numpy as jnp
from jax import lax
from jax.experimental import pallas as pl
from jax.experimental.pallas import tpu as pltpu
from jax.experimental.pallas import tpu_sc as plsc

_TM = 4096
_SPLIT = 50048
_TT = 2176


def _halves_table(aT):
    D, V = aT.shape
    nblk = _SPLIT // _TT

    def body(a1_ref, a2_ref, out_ref):
        out_ref[...] = jnp.concatenate(
            [a1_ref[...], a2_ref[...]], axis=0).T

    return pl.pallas_call(
        body,
        grid=(nblk,),
        in_specs=[
            pl.BlockSpec((D, _TT), lambda i: (0, i)),
            pl.BlockSpec((D, _TT), lambda i: (0, i + nblk)),
        ],
        out_specs=pl.BlockSpec((_TT, 2 * D), lambda i: (i, 0)),
        out_shape=jax.ShapeDtypeStruct((_SPLIT, 2 * D), jnp.float32),
        compiler_params=pltpu.CompilerParams(
            dimension_semantics=("arbitrary",)),
    )(aT, aT)


def _sc_gather(table2, idx):
    _, D2 = table2.shape
    B = idx.shape[0]
    info = plsc.get_sparse_core_info()
    nw = info.num_cores * info.num_subcores
    b_per_w = B // nw
    mesh = plsc.VectorSubcoreMesh(core_axis_name="c", subcore_axis_name="s")

    @functools.partial(
        pl.kernel,
        mesh=mesh,
        out_type=jax.ShapeDtypeStruct((B, D2), jnp.float32),
        scratch_types=[
            pltpu.VMEM((b_per_w,), jnp.int32),
            pltpu.VMEM((b_per_w, D2), jnp.float32),
            pltpu.SemaphoreType.DMA,
        ],
    )
    def gather_kernel(table_hbm, idx_hbm, out_hbm, idx_v, rows_v, sem):
        wid = lax.axis_index("s") * info.num_cores + lax.axis_index("c")
        base = wid * b_per_w
        pltpu.sync_copy(idx_hbm.at[pl.ds(base, b_per_w)], idx_v)
        pltpu.async_copy(table_hbm.at[idx_v], rows_v, sem).wait()
        pltpu.sync_copy(rows_v, out_hbm.at[pl.ds(base, b_per_w)])

    return gather_kernel(table2, idx)


def _gru_proj_t(e2, par, carry, W_ir, b_ir, W_iz, b_iz, W_in, b_in,
                W_hr, W_hz, W_hn, b_hn, WT, b_out):
    B, D2 = e2.shape
    D = D2 // 2
    H = carry.shape[1]
    V = WT.shape[0]
    grid = (pl.cdiv(V, _TM),)

    def body(e2_ref, par_ref, c_ref, wir, bir, wiz, biz, win, bin_, whr, whz,
             whn, bhn, wt, bout, logitsT_ref, newh_ref, h_s, hT_s):
        @pl.when(pl.program_id(0) == 0)
        def _():
            p = par_ref[...]
            e_ = jnp.where(p == 0.0, e2_ref[:, :D], e2_ref[:, D:])
            c = c_ref[...]
            hr = jnp.dot(c, whr[...], preferred_element_type=jnp.float32)
            hz = jnp.dot(c, whz[...], preferred_element_type=jnp.float32)
            hn = jnp.dot(c, whn[...], preferred_element_type=jnp.float32)
            r = jax.nn.sigmoid(
                jnp.dot(e_, wir[...], preferred_element_type=jnp.float32)
                + bir[...] + hr)
            z = jax.nn.sigmoid(
                jnp.dot(e_, wiz[...], preferred_element_type=jnp.float32)
                + biz[...] + hz)
            n = jnp.tanh(
                jnp.dot(e_, win[...], preferred_element_type=jnp.float32)
                + bin_[...] + r * (hn + bhn[...]))
            h = (1.0 - z) * n + z * c
            h_s[...] = h
            hT_s[...] = h.T
            newh_ref[...] = h

        logitsT_ref[...] = (
            jnp.dot(wt[...], hT_s[...], preferred_element_type=jnp.float32)
            + bout[...].T)

    const = lambda shape: pl.BlockSpec(shape, lambda i: (0,) * len(shape))
    out = pl.pallas_call(
        body,
        grid=grid,
        in_specs=[
            const((B, D2)),
            const((B, 1)),
            const((B, H)),
            const((D, H)), const((1, H)),
            const((D, H)), const((1, H)),
            const((D, H)), const((1, H)),
            const((H, H)), const((H, H)), const((H, H)),
            const((1, H)),
            pl.BlockSpec((_TM, H), lambda i: (i, 0)),
            pl.BlockSpec((1, _TM), lambda i: (0, i)),
        ],
        out_specs=[
            pl.BlockSpec((_TM, B), lambda i: (i, 0)),
            const((B, H)),
        ],
        out_shape=[
            jax.ShapeDtypeStruct((V, B), jnp.float32),
            jax.ShapeDtypeStruct((B, H), jnp.float32),
        ],
        scratch_shapes=[pltpu.VMEM((B, H), jnp.float32),
                        pltpu.VMEM((H, B), jnp.float32)],
        compiler_params=pltpu.CompilerParams(
            dimension_semantics=("arbitrary",)),
    )(e2, par, carry, W_ir, b_ir.reshape(1, H), W_iz, b_iz.reshape(1, H),
      W_in, b_in.reshape(1, H), W_hr, W_hz, W_hn, b_hn.reshape(1, H),
      WT, b_out.reshape(1, V))
    return out[0], out[1]


def kernel(x, carry, embed_table, W_ir, b_ir, W_iz, b_iz, W_in, b_in,
           W_hr, W_hz, W_hn, b_hn, W_out, b_out):
    table2 = _halves_table(embed_table.T)
    e2 = _sc_gather(table2, x % _SPLIT)
    par = (x >= _SPLIT).astype(jnp.float32).reshape(-1, 1)
    logitsT, new_h = _gru_proj_t(e2, par, carry, W_ir, b_ir, W_iz, b_iz,
                                 W_in, b_in, W_hr, W_hz, W_hn, b_hn,
                                 W_out.T, b_out)
    return (logitsT.T, new_h)


# speedup vs baseline: 3.4689x; 1.0518x over previous
import functools

import jax
import jax.numpy as jnp
from jax import lax
from jax.experimental import pallas as pl
from jax.experimental.pallas import tpu as pltpu
from jax.experimental.pallas import tpu_sc as plsc

_TM = 4096
_SPLIT = 50176
_TT = 12544


def _halves_table(aT):
    D, V = aT.shape
    nblk = _SPLIT // _TT

    def body(a1_ref, a2_ref, out_ref):
        out_ref[...] = jnp.concatenate(
            [a1_ref[...], a2_ref[...]], axis=0).T

    return pl.pallas_call(
        body,
        grid=(nblk,),
        in_specs=[
            pl.BlockSpec((D, _TT), lambda i: (0, i)),
            pl.BlockSpec((D, _TT), lambda i: (0, i + nblk)),
        ],
        out_specs=pl.BlockSpec((_TT, 2 * D), lambda i: (i, 0)),
        out_shape=jax.ShapeDtypeStruct((_SPLIT, 2 * D), jnp.float32),
        compiler_params=pltpu.CompilerParams(
            dimension_semantics=("arbitrary",)),
    )(aT, aT)


def _sc_gather(table2, idx):
    _, D2 = table2.shape
    B = idx.shape[0]
    info = plsc.get_sparse_core_info()
    nw = info.num_cores * info.num_subcores
    b_per_w = B // nw
    mesh = plsc.VectorSubcoreMesh(core_axis_name="c", subcore_axis_name="s")

    @functools.partial(
        pl.kernel,
        mesh=mesh,
        out_type=jax.ShapeDtypeStruct((B, D2), jnp.float32),
        scratch_types=[
            pltpu.VMEM((b_per_w,), jnp.int32),
            pltpu.VMEM((b_per_w, D2), jnp.float32),
            pltpu.SemaphoreType.DMA,
        ],
    )
    def gather_kernel(table_hbm, idx_hbm, out_hbm, idx_v, rows_v, sem):
        wid = lax.axis_index("s") * info.num_cores + lax.axis_index("c")
        base = wid * b_per_w
        pltpu.sync_copy(idx_hbm.at[pl.ds(base, b_per_w)], idx_v)
        for j in range(b_per_w // 16):
            v = idx_v[pl.ds(16 * j, 16)]
            idx_v[pl.ds(16 * j, 16)] = jnp.where(v >= _SPLIT, v - _SPLIT, v)
        pltpu.async_copy(table_hbm.at[idx_v], rows_v, sem).wait()
        pltpu.sync_copy(rows_v, out_hbm.at[pl.ds(base, b_per_w)])

    return gather_kernel(table2, idx)


def _gru_proj_t(e2, par, carry, W_ir, b_ir, W_iz, b_iz, W_in, b_in,
                W_hr, W_hz, W_hn, b_hn, WT, b_out):
    B, D2 = e2.shape
    D = D2 // 2
    H = carry.shape[1]
    V = WT.shape[0]
    grid = (pl.cdiv(V, _TM),)

    def body(e2_ref, par_ref, c_ref, wir, bir, wiz, biz, win, bin_, whr, whz,
             whn, bhn, wt, bout, logitsT_ref, newh_ref, h_s, hT_s):
        @pl.when(pl.program_id(0) == 0)
        def _():
            p = par_ref[...]
            e_ = jnp.where(p == 0.0, e2_ref[:, :D], e2_ref[:, D:])
            c = c_ref[...]
            hr = jnp.dot(c, whr[...], preferred_element_type=jnp.float32)
            hz = jnp.dot(c, whz[...], preferred_element_type=jnp.float32)
            hn = jnp.dot(c, whn[...], preferred_element_type=jnp.float32)
            r = jax.nn.sigmoid(
                jnp.dot(e_, wir[...], preferred_element_type=jnp.float32)
                + bir[...] + hr)
            z = jax.nn.sigmoid(
                jnp.dot(e_, wiz[...], preferred_element_type=jnp.float32)
                + biz[...] + hz)
            n = jnp.tanh(
                jnp.dot(e_, win[...], preferred_element_type=jnp.float32)
                + bin_[...] + r * (hn + bhn[...]))
            h = (1.0 - z) * n + z * c
            h_s[...] = h
            hT_s[...] = h.T
            newh_ref[...] = h

        logitsT_ref[...] = (
            jnp.dot(wt[...], hT_s[...], preferred_element_type=jnp.float32)
            + bout[...].T)

    const = lambda shape: pl.BlockSpec(shape, lambda i: (0,) * len(shape))
    out = pl.pallas_call(
        body,
        grid=grid,
        in_specs=[
            const((B, D2)),
            const((B, 1)),
            const((B, H)),
            const((D, H)), const((1, H)),
            const((D, H)), const((1, H)),
            const((D, H)), const((1, H)),
            const((H, H)), const((H, H)), const((H, H)),
            const((1, H)),
            pl.BlockSpec((_TM, H), lambda i: (i, 0)),
            pl.BlockSpec((1, _TM), lambda i: (0, i)),
        ],
        out_specs=[
            pl.BlockSpec((_TM, B), lambda i: (i, 0)),
            const((B, H)),
        ],
        out_shape=[
            jax.ShapeDtypeStruct((V, B), jnp.float32),
            jax.ShapeDtypeStruct((B, H), jnp.float32),
        ],
        scratch_shapes=[pltpu.VMEM((B, H), jnp.float32),
                        pltpu.VMEM((H, B), jnp.float32)],
        compiler_params=pltpu.CompilerParams(
            dimension_semantics=("arbitrary",)),
    )(e2, par, carry, W_ir, b_ir.reshape(1, H), W_iz, b_iz.reshape(1, H),
      W_in, b_in.reshape(1, H), W_hr, W_hz, W_hn, b_hn.reshape(1, H),
      WT, b_out.reshape(1, V))
    return out[0], out[1]


def kernel(x, carry, embed_table, W_ir, b_ir, W_iz, b_iz, W_in, b_in,
           W_hr, W_hz, W_hn, b_hn, W_out, b_out):
    table2 = _halves_table(embed_table.T)
    e2 = _sc_gather(table2, x)
    par = (x >= _SPLIT).astype(jnp.float32).reshape(-1, 1)
    logitsT, new_h = _gru_proj_t(e2, par, carry, W_ir, b_ir, W_iz, b_iz,
                                 W_in, b_in, W_hr, W_hz, W_hn, b_hn,
                                 W_out.T, b_out)
    return (logitsT.T, new_h)
